# baseline (speedup 1.0000x reference)
import functools

import jax
import jax.numpy as jnp
from jax import lax
from jax.experimental import pallas as pl
from jax.experimental.pallas import tpu as pltpu
from jax.experimental.pallas import tpu_sc as plsc

_B = 16
_N = 20000
_FIGSIZE = 416.0
_IOU_THRESH = 0.2
_NC = 2
_NS = 16
_NW = _NC * _NS
_BOX_W = 7
_CHUNK = _N // 2
_CHUNK_W = _CHUNK * _BOX_W
_ITERS = _CHUNK // 16


def _sc_partial_max(boxes_flat, gt):
    mesh = plsc.VectorSubcoreMesh(core_axis_name="c", subcore_axis_name="s")

    @functools.partial(
        pl.kernel,
        mesh=mesh,
        out_type=jax.ShapeDtypeStruct((_NW, 16), jnp.float32),
        compiler_params=pltpu.CompilerParams(needs_layout_passes=False),
        scratch_types=[
            pltpu.VMEM((_CHUNK_W,), jnp.float32),
            pltpu.VMEM((_B * 4,), jnp.float32),
            pltpu.VMEM((16,), jnp.float32),
        ],
    )
    def k(boxes_hbm, gt_hbm, out_hbm, boxes_v, gt_v, acc_v):
        cid = lax.axis_index("c")
        sid = lax.axis_index("s")
        wid = sid * _NC + cid
        b = wid // 2

        pltpu.sync_copy(gt_hbm, gt_v)
        pltpu.sync_copy(
            boxes_hbm.at[pl.ds(wid * _CHUNK_W, _CHUNK_W)], boxes_v)

        gbase = jnp.full((16,), b * 4, jnp.int32)
        gx1v = plsc.load_gather(gt_v, [gbase])
        gy1v = plsc.load_gather(gt_v, [gbase + 1])
        gx2v = plsc.load_gather(gt_v, [gbase + 2])
        gy2v = plsc.load_gather(gt_v, [gbase + 3])
        agv = (gx2v - gx1v) * (gy2v - gy1v)

        f_base = lax.iota(jnp.int32, 16) * _BOX_W
        neg = jnp.full((16,), -1e9, jnp.float32)

        def body(i, acc):
            idx = f_base + i * (16 * _BOX_W)
            x = plsc.load_gather(boxes_v, [idx])
            y = plsc.load_gather(boxes_v, [idx + 1])
            w = plsc.load_gather(boxes_v, [idx + 2])
            h = plsc.load_gather(boxes_v, [idx + 3])
            c = plsc.load_gather(boxes_v, [idx + 6])
            xs = x * _FIGSIZE
            ys = y * _FIGSIZE
            wh = w * (0.5 * _FIGSIZE)
            hh = h * (0.5 * _FIGSIZE)
            bx1 = xs - wh
            bx2 = xs + wh
            by1 = ys - hh
            by2 = ys + hh
            ix1 = jnp.maximum(bx1, gx1v)
            iy1 = jnp.maximum(by1, gy1v)
            ix2 = jnp.minimum(bx2, gx2v)
            iy2 = jnp.minimum(by2, gy2v)
            dx = jnp.maximum(ix2 - ix1, 0.0)
            dy = jnp.maximum(iy2 - iy1, 0.0)
            inter = dx * dy
            area_b = (wh * hh) * 4.0
            iou = inter / ((area_b + agv) - inter)
            val = jnp.where(c == 0.0, iou, neg)
            return jnp.maximum(acc, val)

        acc = lax.fori_loop(0, _ITERS, body, neg)
        acc_v[...] = acc
        pltpu.sync_copy(acc_v, out_hbm.at[wid])

    return k(boxes_flat, gt.reshape(-1))


def _tc_finish_body(p_ref, mp_ref, dl_ref):
    p = p_ref[...]
    m = jnp.max(p, axis=1, keepdims=True)
    m = jnp.where(m >= _IOU_THRESH, m, 0.0)
    mp_ref[...] = m
    dl_ref[...] = jnp.sum(m).reshape(1, 1) * (1.0 / _B)


def kernel(boxes, gt):
    boxes_flat = boxes.reshape(-1)
    partials = _sc_partial_max(boxes_flat, gt)
    mp, dl = pl.pallas_call(
        _tc_finish_body,
        out_shape=[
            jax.ShapeDtypeStruct((_B, 1), jnp.float32),
            jax.ShapeDtypeStruct((1, 1), jnp.float32),
        ],
    )(partials.reshape(_B, 2 * 16))
    return dl[0, 0], mp.reshape(_B)


# speedup vs baseline: 6.4373x; 6.4373x over previous
import functools

import jax
import jax.numpy as jnp
from jax import lax
from jax.experimental import pallas as pl
from jax.experimental.pallas import tpu as pltpu
from jax.experimental.pallas import tpu_sc as plsc

_B = 16
_N = 20000
_IOU_THRESH = 0.2
_NC = 2
_NS = 16
_NW = _NC * _NS
_NT = 156
_TAIL = _NT * 128
_FIELDS = (0, 1, 2, 3, 6)


def _sc_partial_max(boxes_t, gt):
    mesh = plsc.VectorSubcoreMesh(core_axis_name="c", subcore_axis_name="s")

    @functools.partial(
        pl.kernel,
        mesh=mesh,
        out_type=jax.ShapeDtypeStruct((_NW, 2, 128), jnp.float32),
        compiler_params=pltpu.CompilerParams(
            needs_layout_passes=False, use_tc_tiling_on_sc=True),
        scratch_types=(
            [pltpu.VMEM((2, _B, 128), jnp.float32) for _ in _FIELDS]
            + [
                pltpu.VMEM((_B * 4,), jnp.float32),
                pltpu.VMEM((_B, 16), jnp.float32),
                pltpu.VMEM((2, 128), jnp.float32),
                pltpu.SemaphoreType.DMA,
                pltpu.SemaphoreType.DMA,
            ]
        ),
    )
    def k(boxes_hbm, gt_hbm, out_hbm, xv, yv, wv, hv, cv, gt_v, acc_v,
          flat_v, sem0, sem1):
        cid = lax.axis_index("c")
        sid = lax.axis_index("s")
        wid = sid * _NC + cid

        pltpu.sync_copy(gt_hbm, gt_v)

        bufs = (xv, yv, wv, hv, cv)
        sems = (sem0, sem1)

        def start_tile(t, slot):
            for f, buf in zip(_FIELDS, bufs):
                pltpu.async_copy(
                    boxes_hbm.at[f, :, pl.ds(t * 128, 128)],
                    buf.at[slot], sems[slot])

        def wait_tile(t, slot):
            for f, buf in zip(_FIELDS, bufs):
                pltpu.make_async_copy(
                    boxes_hbm.at[f, :, pl.ds(t * 128, 128)],
                    buf.at[slot], sems[slot]).wait()

        neg = jnp.full((16,), -1e9, jnp.float32)

        def init_acc(b, _):
            acc_v[b] = neg
            return 0

        lax.fori_loop(0, _B, init_acc, 0)

        n_tiles = (_NT - wid + _NW - 1) // _NW
        start_tile(wid, 0)

        def process_tile(i, slot):
            t = wid + i * _NW

            @pl.when(i + 1 < n_tiles)
            def _():
                start_tile(t + _NW, 1 - slot)

            wait_tile(t, slot)

            def img_body(b, _):
                gbase = jnp.full((16,), b * 4, jnp.int32)
                gx1 = plsc.load_gather(gt_v, [gbase])
                gy1 = plsc.load_gather(gt_v, [gbase + 1])
                gx2 = plsc.load_gather(gt_v, [gbase + 2])
                gy2 = plsc.load_gather(gt_v, [gbase + 3])
                agv = (gx2 - gx1) * (gy2 - gy1)
                acc = acc_v[b]
                for c in range(8):
                    sl = pl.ds(c * 16, 16)
                    x = xv[slot, b, sl]
                    y = yv[slot, b, sl]
                    w = wv[slot, b, sl]
                    h = hv[slot, b, sl]
                    cl = cv[slot, b, sl]
                    wh = w * 0.5
                    hh = h * 0.5
                    bx1 = x - wh
                    bx2 = x + wh
                    by1 = y - hh
                    by2 = y + hh
                    ix1 = jnp.maximum(bx1, gx1)
                    iy1 = jnp.maximum(by1, gy1)
                    ix2 = jnp.minimum(bx2, gx2)
                    iy2 = jnp.minimum(by2, gy2)
                    dx = jnp.maximum(ix2 - ix1, 0.0)
                    dy = jnp.maximum(iy2 - iy1, 0.0)
                    inter = dx * dy
                    area_b = w * h
                    iou = inter / ((area_b + agv) - inter)
                    val = jnp.where(cl == 0.0, iou, neg)
                    acc = jnp.maximum(acc, val)
                acc_v[b] = acc
                return 0

            lax.fori_loop(0, _B, img_body, 0)

        def pair_body(p, _):
            for s in (0, 1):
                i = p * 2 + s

                @pl.when(i < n_tiles)
                def _(i=i, s=s):
                    process_tile(i, s)
            return 0

        lax.fori_loop(0, (n_tiles + 1) // 2, pair_body, 0)

        for b in range(_B):
            flat_v[b // 8, pl.ds((b % 8) * 16, 16)] = acc_v[b]
        pltpu.sync_copy(flat_v, out_hbm.at[wid])

    return k(boxes_t, gt.reshape(-1))


def _tc_finish_body(p_ref, t_ref, gt_ref, mp_ref, dl_ref):
    x, y, w, h, cl = (t_ref[f] for f in (0, 1, 2, 3, 6))
    gx1 = gt_ref[:, 0:1]
    gy1 = gt_ref[:, 1:2]
    gx2 = gt_ref[:, 2:3]
    gy2 = gt_ref[:, 3:4]
    wh = w * 0.5
    hh = h * 0.5
    ix1 = jnp.maximum(x - wh, gx1)
    iy1 = jnp.maximum(y - hh, gy1)
    ix2 = jnp.minimum(x + wh, gx2)
    iy2 = jnp.minimum(y + hh, gy2)
    inter = jnp.maximum(ix2 - ix1, 0.0) * jnp.maximum(iy2 - iy1, 0.0)
    area_g = (gx2 - gx1) * (gy2 - gy1)
    iou = inter / ((w * h + area_g) - inter)
    tval = jnp.where(cl == 0.0, iou, -1e9)
    tmax = jnp.max(tval, axis=1)

    p = jnp.max(p_ref[...], axis=0)
    pmax = jnp.max(p.reshape(2, 8, 16), axis=2).reshape(16)

    m = jnp.maximum(pmax, tmax)
    m = jnp.where(m >= _IOU_THRESH, m, 0.0)
    mp_ref[...] = m.reshape(_B, 1)
    dl_ref[...] = jnp.sum(m).reshape(1, 1) * (1.0 / _B)


def kernel(boxes, gt):
    boxes_t = jnp.transpose(boxes, (2, 0, 1))
    gt_n = gt * (1.0 / 416.0)
    partials = _sc_partial_max(boxes_t, gt_n)
    tail = lax.slice(boxes_t, (0, 0, _TAIL), (7, _B, _N))
    mp, dl = pl.pallas_call(
        _tc_finish_body,
        out_shape=[
            jax.ShapeDtypeStruct((_B, 1), jnp.float32),
            jax.ShapeDtypeStruct((1, 1), jnp.float32),
        ],
    )(partials, tail, gt_n)
    return dl[0, 0], mp.reshape(_B)


# speedup vs baseline: 6.7913x; 1.0550x over previous
import functools

import jax
import jax.numpy as jnp
from jax import lax
from jax.experimental import pallas as pl
from jax.experimental.pallas import tpu as pltpu
from jax.experimental.pallas import tpu_sc as plsc

_B = 16
_N = 20000
_IOU_THRESH = 0.2
_NC = 2
_NS = 16
_NW = _NC * _NS
_NT = 156
_TAIL = _NT * 128
_FIELDS = (0, 1, 2, 3, 6)


def _sc_partial_max(boxes_t, gt8):
    mesh = plsc.VectorSubcoreMesh(core_axis_name="c", subcore_axis_name="s")

    @functools.partial(
        pl.kernel,
        mesh=mesh,
        out_type=jax.ShapeDtypeStruct((_NW, 2, 128), jnp.float32),
        compiler_params=pltpu.CompilerParams(
            needs_layout_passes=False, use_tc_tiling_on_sc=True),
        scratch_types=(
            [pltpu.VMEM((2, _B, 128), jnp.float32) for _ in _FIELDS]
            + [
                pltpu.VMEM((_B, 128), jnp.float32),
                pltpu.VMEM((_B, 16), jnp.float32),
                pltpu.VMEM((2, 128), jnp.float32),
                pltpu.SemaphoreType.DMA,
                pltpu.SemaphoreType.DMA,
            ]
        ),
    )
    def k(boxes_hbm, gt_hbm, out_hbm, xv, yv, wv, hv, cv, gt_v, acc_v,
          flat_v, sem0, sem1):
        cid = lax.axis_index("c")
        sid = lax.axis_index("s")
        wid = sid * _NC + cid

        pltpu.sync_copy(gt_hbm, gt_v)

        bufs = (xv, yv, wv, hv, cv)
        sems = (sem0, sem1)

        def start_tile(t, slot):
            for f, buf in zip(_FIELDS, bufs):
                pltpu.async_copy(
                    boxes_hbm.at[f, :, pl.ds(t * 128, 128)],
                    buf.at[slot], sems[slot])

        def wait_tile(t, slot):
            for f, buf in zip(_FIELDS, bufs):
                pltpu.make_async_copy(
                    boxes_hbm.at[f, :, pl.ds(t * 128, 128)],
                    buf.at[slot], sems[slot]).wait()

        neg = jnp.full((16,), -1e9, jnp.float32)
        c0 = jnp.full((16,), 0, jnp.int32)
        c1 = jnp.full((16,), 1, jnp.int32)
        c2 = jnp.full((16,), 2, jnp.int32)
        c3 = jnp.full((16,), 3, jnp.int32)

        def init_acc(b, _):
            acc_v[b] = neg
            return 0

        lax.fori_loop(0, _B, init_acc, 0)

        n_tiles = (_NT - wid + _NW - 1) // _NW
        start_tile(wid, 0)

        def process_tile(i, slot):
            t = wid + i * _NW

            @pl.when(i + 1 < n_tiles)
            def _():
                start_tile(t + _NW, 1 - slot)

            wait_tile(t, slot)

            def img_body(b, _):
                brow = jnp.full((16,), b, jnp.int32)
                gx1 = plsc.load_gather(gt_v, [brow, c0])
                gy1 = plsc.load_gather(gt_v, [brow, c1])
                gx2 = plsc.load_gather(gt_v, [brow, c2])
                gy2 = plsc.load_gather(gt_v, [brow, c3])
                agv = (gx2 - gx1) * (gy2 - gy1)
                acc = acc_v[b]
                for c in range(8):
                    sl = pl.ds(c * 16, 16)
                    x = xv[slot, b, sl]
                    y = yv[slot, b, sl]
                    w = wv[slot, b, sl]
                    h = hv[slot, b, sl]
                    cl = cv[slot, b, sl]
                    wh = w * 0.5
                    hh = h * 0.5
                    bx1 = x - wh
                    bx2 = x + wh
                    by1 = y - hh
                    by2 = y + hh
                    ix1 = jnp.maximum(bx1, gx1)
                    iy1 = jnp.maximum(by1, gy1)
                    ix2 = jnp.minimum(bx2, gx2)
                    iy2 = jnp.minimum(by2, gy2)
                    dx = jnp.maximum(ix2 - ix1, 0.0)
                    dy = jnp.maximum(iy2 - iy1, 0.0)
                    inter = dx * dy
                    area_b = w * h
                    iou = inter / ((area_b + agv) - inter)
                    val = jnp.where(cl == 0.0, iou, neg)
                    acc = jnp.maximum(acc, val)
                acc_v[b] = acc
                return 0

            lax.fori_loop(0, _B, img_body, 0)

        def pair_body(p, _):
            for s in (0, 1):
                i = p * 2 + s

                @pl.when(i < n_tiles)
                def _(i=i, s=s):
                    process_tile(i, s)
            return 0

        lax.fori_loop(0, (n_tiles + 1) // 2, pair_body, 0)

        for b in range(_B):
            flat_v[b // 8, pl.ds((b % 8) * 16, 16)] = acc_v[b]
        pltpu.sync_copy(flat_v, out_hbm.at[wid])

    return k(boxes_t, gt8)


def _tc_finish_body(p_ref, t_ref, g_ref, mp_ref, dl_ref):
    gx1 = g_ref[:, 0:1]
    gy1 = g_ref[:, 1:2]
    gx2 = g_ref[:, 2:3]
    gy2 = g_ref[:, 3:4]

    x, y, w, h, cl = (t_ref[f] for f in (0, 1, 2, 3, 6))
    wh = w * 0.5
    hh = h * 0.5
    ix1 = jnp.maximum(x - wh, gx1)
    iy1 = jnp.maximum(y - hh, gy1)
    ix2 = jnp.minimum(x + wh, gx2)
    iy2 = jnp.minimum(y + hh, gy2)
    inter = jnp.maximum(ix2 - ix1, 0.0) * jnp.maximum(iy2 - iy1, 0.0)
    area_g = (gx2 - gx1) * (gy2 - gy1)
    iou = inter / ((w * h + area_g) - inter)
    tval = jnp.where(cl == 0.0, iou, -1e9)
    tmax = jnp.max(tval, axis=1)

    p = jnp.max(p_ref[...], axis=0)
    pmax = jnp.max(p.reshape(2, 8, 16), axis=2).reshape(16)

    m = jnp.maximum(pmax, tmax)
    m = jnp.where(m >= _IOU_THRESH, m, 0.0)
    mp_ref[...] = m.reshape(1, _B)
    dl_ref[...] = jnp.sum(m).reshape(1, 1) * (1.0 / _B)


def kernel(boxes, gt):
    boxes_t = jnp.transpose(boxes, (2, 0, 1))
    gt_n = gt * (1.0 / 416.0)
    gt8 = jnp.zeros((_B, 128), jnp.float32).at[:, :4].set(gt_n)
    partials = _sc_partial_max(boxes_t, gt8)
    tail = lax.slice(boxes_t, (0, 0, _TAIL), (7, _B, _N))
    mp, dl = pl.pallas_call(
        _tc_finish_body,
        out_shape=[
            jax.ShapeDtypeStruct((1, _B), jnp.float32),
            jax.ShapeDtypeStruct((1, 1), jnp.float32),
        ],
    )(partials, tail, gt8)
    return dl[0, 0], mp.reshape(_B)


# speedup vs baseline: 6.8716x; 1.0118x over previous
import functools

import jax
import jax.numpy as jnp
from jax import lax
from jax.experimental import pallas as pl
from jax.experimental.pallas import tpu as pltpu
from jax.experimental.pallas import tpu_sc as plsc

_B = 16
_N = 20000
_IOU_THRESH = 0.2
_NC = 2
_NS = 16
_NW = _NC * _NS
_NT = 156
_TAIL = _NT * 128
_FIELDS = (0, 1, 2, 3, 6)


def _sc_partial_max(boxes_t, gt8):
    mesh = plsc.VectorSubcoreMesh(core_axis_name="c", subcore_axis_name="s")

    @functools.partial(
        pl.kernel,
        mesh=mesh,
        out_type=jax.ShapeDtypeStruct((_NW, 2, 128), jnp.float32),
        compiler_params=pltpu.CompilerParams(
            needs_layout_passes=False, use_tc_tiling_on_sc=True),
        scratch_types=[
            pltpu.VMEM((2, 4, _B, 128), jnp.float32),
            pltpu.VMEM((2, _B, 128), jnp.float32),
            pltpu.VMEM((_B, 128), jnp.float32),
            pltpu.VMEM((_B, 16), jnp.float32),
            pltpu.VMEM((2, 128), jnp.float32),
            pltpu.SemaphoreType.DMA,
            pltpu.SemaphoreType.DMA,
        ],
    )
    def k(boxes_hbm, gt_hbm, out_hbm, bv, cv, gt_v, acc_v,
          flat_v, sem0, sem1):
        cid = lax.axis_index("c")
        sid = lax.axis_index("s")
        wid = sid * _NC + cid

        pltpu.sync_copy(gt_hbm, gt_v)

        sems = (sem0, sem1)

        def start_tile(t, slot):
            pltpu.async_copy(
                boxes_hbm.at[pl.ds(0, 4), :, pl.ds(t * 128, 128)],
                bv.at[slot], sems[slot])
            pltpu.async_copy(
                boxes_hbm.at[6, :, pl.ds(t * 128, 128)],
                cv.at[slot], sems[slot])

        def wait_tile(t, slot):
            pltpu.make_async_copy(
                boxes_hbm.at[pl.ds(0, 4), :, pl.ds(t * 128, 128)],
                bv.at[slot], sems[slot]).wait()
            pltpu.make_async_copy(
                boxes_hbm.at[6, :, pl.ds(t * 128, 128)],
                cv.at[slot], sems[slot]).wait()

        neg = jnp.full((16,), -1e9, jnp.float32)
        c0 = jnp.full((16,), 0, jnp.int32)
        c1 = jnp.full((16,), 1, jnp.int32)
        c2 = jnp.full((16,), 2, jnp.int32)
        c3 = jnp.full((16,), 3, jnp.int32)

        def init_acc(b, _):
            acc_v[b] = neg
            return 0

        lax.fori_loop(0, _B, init_acc, 0)

        n_tiles = (_NT - wid + _NW - 1) // _NW
        start_tile(wid, 0)

        def process_tile(i, slot):
            t = wid + i * _NW

            @pl.when(i + 1 < n_tiles)
            def _():
                start_tile(t + _NW, 1 - slot)

            wait_tile(t, slot)

            def img_body(b, _):
                brow = jnp.full((16,), b, jnp.int32)
                gx1 = plsc.load_gather(gt_v, [brow, c0])
                gy1 = plsc.load_gather(gt_v, [brow, c1])
                gx2 = plsc.load_gather(gt_v, [brow, c2])
                gy2 = plsc.load_gather(gt_v, [brow, c3])
                agv = (gx2 - gx1) * (gy2 - gy1)

                def sub_body(c, acc):
                    sl = pl.ds(c * 16, 16)
                    x = bv[slot, 0, b, sl]
                    y = bv[slot, 1, b, sl]
                    w = bv[slot, 2, b, sl]
                    h = bv[slot, 3, b, sl]
                    cl = cv[slot, b, sl]
                    wh = w * 0.5
                    hh = h * 0.5
                    bx1 = x - wh
                    bx2 = x + wh
                    by1 = y - hh
                    by2 = y + hh
                    ix1 = jnp.maximum(bx1, gx1)
                    iy1 = jnp.maximum(by1, gy1)
                    ix2 = jnp.minimum(bx2, gx2)
                    iy2 = jnp.minimum(by2, gy2)
                    dx = jnp.maximum(ix2 - ix1, 0.0)
                    dy = jnp.maximum(iy2 - iy1, 0.0)
                    inter = dx * dy
                    area_b = w * h
                    iou = inter / ((area_b + agv) - inter)
                    val = jnp.where(cl == 0.0, iou, neg)
                    return jnp.maximum(acc, val)

                acc_v[b] = lax.fori_loop(0, 8, sub_body, acc_v[b])
                return 0

            lax.fori_loop(0, _B, img_body, 0)

        def pair_body(p, _):
            for s in (0, 1):
                i = p * 2 + s

                @pl.when(i < n_tiles)
                def _(i=i, s=s):
                    process_tile(i, s)
            return 0

        lax.fori_loop(0, (n_tiles + 1) // 2, pair_body, 0)

        def out_body(b, _):
            flat_v[b // 8, pl.ds((b % 8) * 16, 16)] = acc_v[b]
            return 0

        lax.fori_loop(0, _B, out_body, 0)
        pltpu.sync_copy(flat_v, out_hbm.at[wid])

    return k(boxes_t, gt8)


def _tc_finish_body(p_ref, t_ref, g_ref, mp_ref, dl_ref):
    gx1 = g_ref[:, 0:1]
    gy1 = g_ref[:, 1:2]
    gx2 = g_ref[:, 2:3]
    gy2 = g_ref[:, 3:4]

    x, y, w, h, cl = (t_ref[f] for f in (0, 1, 2, 3, 6))
    wh = w * 0.5
    hh = h * 0.5
    ix1 = jnp.maximum(x - wh, gx1)
    iy1 = jnp.maximum(y - hh, gy1)
    ix2 = jnp.minimum(x + wh, gx2)
    iy2 = jnp.minimum(y + hh, gy2)
    inter = jnp.maximum(ix2 - ix1, 0.0) * jnp.maximum(iy2 - iy1, 0.0)
    area_g = (gx2 - gx1) * (gy2 - gy1)
    iou = inter / ((w * h + area_g) - inter)
    tval = jnp.where(cl == 0.0, iou, -1e9)
    tmax = jnp.max(tval, axis=1)

    p = jnp.max(p_ref[...], axis=0)
    pmax = jnp.max(p.reshape(2, 8, 16), axis=2).reshape(16)

    m = jnp.maximum(pmax, tmax)
    m = jnp.where(m >= _IOU_THRESH, m, 0.0)
    mp_ref[...] = m.reshape(1, _B)
    dl_ref[...] = jnp.sum(m).reshape(1, 1) * (1.0 / _B)


def kernel(boxes, gt):
    boxes_t = jnp.transpose(boxes, (2, 0, 1))
    gt_n = gt * (1.0 / 416.0)
    gt8 = jnp.zeros((_B, 128), jnp.float32).at[:, :4].set(gt_n)
    partials = _sc_partial_max(boxes_t, gt8)
    tail = lax.slice(boxes_t, (0, 0, _TAIL), (7, _B, _N))
    mp, dl = pl.pallas_call(
        _tc_finish_body,
        out_shape=[
            jax.ShapeDtypeStruct((1, _B), jnp.float32),
            jax.ShapeDtypeStruct((1, 1), jnp.float32),
        ],
    )(partials, tail, gt8)
    return dl[0, 0], mp.reshape(_B)


# speedup vs baseline: 6.8944x; 1.0033x over previous
import functools

import jax
import jax.numpy as jnp
from jax import lax
from jax.experimental import pallas as pl
from jax.experimental.pallas import tpu as pltpu
from jax.experimental.pallas import tpu_sc as plsc

_B = 16
_N = 20000
_IOU_THRESH = 0.2
_NC = 2
_NS = 16
_NW = _NC * _NS
_NT = 156
_TAIL = _NT * 128
_FIELDS = (0, 1, 2, 3, 6)


def _sc_partial_max(boxes_t, gt8):
    mesh = plsc.VectorSubcoreMesh(core_axis_name="c", subcore_axis_name="s")

    @functools.partial(
        pl.kernel,
        mesh=mesh,
        out_type=jax.ShapeDtypeStruct((_NW, 2, 128), jnp.float32),
        compiler_params=pltpu.CompilerParams(
            needs_layout_passes=False, use_tc_tiling_on_sc=True,
            skip_device_barrier=True),
        scratch_types=[
            pltpu.VMEM((2, 4, _B, 128), jnp.float32),
            pltpu.VMEM((2, _B, 128), jnp.float32),
            pltpu.VMEM((_B, 128), jnp.float32),
            pltpu.VMEM((_B, 16), jnp.float32),
            pltpu.VMEM((2, 128), jnp.float32),
            pltpu.SemaphoreType.DMA,
            pltpu.SemaphoreType.DMA,
        ],
    )
    def k(boxes_hbm, gt_hbm, out_hbm, bv, cv, gt_v, acc_v,
          flat_v, sem0, sem1):
        cid = lax.axis_index("c")
        sid = lax.axis_index("s")
        wid = sid * _NC + cid

        pltpu.sync_copy(gt_hbm, gt_v)

        sems = (sem0, sem1)

        def start_tile(t, slot):
            pltpu.async_copy(
                boxes_hbm.at[pl.ds(0, 4), :, pl.ds(t * 128, 128)],
                bv.at[slot], sems[slot])
            pltpu.async_copy(
                boxes_hbm.at[6, :, pl.ds(t * 128, 128)],
                cv.at[slot], sems[slot])

        def wait_tile(t, slot):
            pltpu.make_async_copy(
                boxes_hbm.at[pl.ds(0, 4), :, pl.ds(t * 128, 128)],
                bv.at[slot], sems[slot]).wait()
            pltpu.make_async_copy(
                boxes_hbm.at[6, :, pl.ds(t * 128, 128)],
                cv.at[slot], sems[slot]).wait()

        neg = jnp.full((16,), -1e9, jnp.float32)
        c0 = jnp.full((16,), 0, jnp.int32)
        c1 = jnp.full((16,), 1, jnp.int32)
        c2 = jnp.full((16,), 2, jnp.int32)
        c3 = jnp.full((16,), 3, jnp.int32)

        def init_acc(b, _):
            acc_v[b] = neg
            return 0

        lax.fori_loop(0, _B, init_acc, 0)

        n_tiles = (_NT - wid + _NW - 1) // _NW
        start_tile(wid, 0)

        def process_tile(i, slot):
            t = wid + i * _NW

            @pl.when(i + 1 < n_tiles)
            def _():
                start_tile(t + _NW, 1 - slot)

            wait_tile(t, slot)

            def img_body(b, _):
                brow = jnp.full((16,), b, jnp.int32)
                gx1 = plsc.load_gather(gt_v, [brow, c0])
                gy1 = plsc.load_gather(gt_v, [brow, c1])
                gx2 = plsc.load_gather(gt_v, [brow, c2])
                gy2 = plsc.load_gather(gt_v, [brow, c3])
                agv = (gx2 - gx1) * (gy2 - gy1)

                def sub_body(c, acc):
                    sl = pl.ds(c * 16, 16)
                    x = bv[slot, 0, b, sl]
                    y = bv[slot, 1, b, sl]
                    w = bv[slot, 2, b, sl]
                    h = bv[slot, 3, b, sl]
                    cl = cv[slot, b, sl]
                    wh = w * 0.5
                    hh = h * 0.5
                    bx1 = x - wh
                    bx2 = x + wh
                    by1 = y - hh
                    by2 = y + hh
                    ix1 = jnp.maximum(bx1, gx1)
                    iy1 = jnp.maximum(by1, gy1)
                    ix2 = jnp.minimum(bx2, gx2)
                    iy2 = jnp.minimum(by2, gy2)
                    dx = jnp.maximum(ix2 - ix1, 0.0)
                    dy = jnp.maximum(iy2 - iy1, 0.0)
                    inter = dx * dy
                    area_b = w * h
                    iou = inter / ((area_b + agv) - inter)
                    val = jnp.where(cl == 0.0, iou, neg)
                    return jnp.maximum(acc, val)

                acc_v[b] = lax.fori_loop(0, 8, sub_body, acc_v[b])
                return 0

            lax.fori_loop(0, _B, img_body, 0)

        def pair_body(p, _):
            for s in (0, 1):
                i = p * 2 + s

                @pl.when(i < n_tiles)
                def _(i=i, s=s):
                    process_tile(i, s)
            return 0

        lax.fori_loop(0, (n_tiles + 1) // 2, pair_body, 0)

        def out_body(b, _):
            flat_v[b // 8, pl.ds((b % 8) * 16, 16)] = acc_v[b]
            return 0

        lax.fori_loop(0, _B, out_body, 0)
        pltpu.sync_copy(flat_v, out_hbm.at[wid])

    return k(boxes_t, gt8)


def _tc_finish_body(p_ref, t_ref, g_ref, mp_ref, dl_ref):
    gx1 = g_ref[:, 0:1]
    gy1 = g_ref[:, 1:2]
    gx2 = g_ref[:, 2:3]
    gy2 = g_ref[:, 3:4]

    x, y, w, h, cl = (t_ref[f] for f in (0, 1, 2, 3, 6))
    wh = w * 0.5
    hh = h * 0.5
    ix1 = jnp.maximum(x - wh, gx1)
    iy1 = jnp.maximum(y - hh, gy1)
    ix2 = jnp.minimum(x + wh, gx2)
    iy2 = jnp.minimum(y + hh, gy2)
    inter = jnp.maximum(ix2 - ix1, 0.0) * jnp.maximum(iy2 - iy1, 0.0)
    area_g = (gx2 - gx1) * (gy2 - gy1)
    iou = inter / ((w * h + area_g) - inter)
    tval = jnp.where(cl == 0.0, iou, -1e9)
    tmax = jnp.max(tval, axis=1)

    p = jnp.max(p_ref[...], axis=0)
    pmax = jnp.max(p.reshape(2, 8, 16), axis=2).reshape(16)

    m = jnp.maximum(pmax, tmax)
    m = jnp.where(m >= _IOU_THRESH, m, 0.0)
    mp_ref[...] = m.reshape(1, _B)
    dl_ref[...] = jnp.sum(m).reshape(1, 1) * (1.0 / _B)


def kernel(boxes, gt):
    boxes_t = jnp.transpose(boxes, (2, 0, 1))
    gt_n = gt * (1.0 / 416.0)
    gt8 = jnp.zeros((_B, 128), jnp.float32).at[:, :4].set(gt_n)
    partials = _sc_partial_max(boxes_t, gt8)
    tail = lax.slice(boxes_t, (0, 0, _TAIL), (7, _B, _N))
    mp, dl = pl.pallas_call(
        _tc_finish_body,
        out_shape=[
            jax.ShapeDtypeStruct((1, _B), jnp.float32),
            jax.ShapeDtypeStruct((1, 1), jnp.float32),
        ],
    )(partials, tail, gt8)
    return dl[0, 0], mp.reshape(_B)
